# baseline (speedup 1.0000x reference)
import jax
import jax.numpy as jnp
from jax import lax
from jax.experimental import pallas as pl
from jax.experimental.pallas import tpu as pltpu
from jax.experimental.pallas import tpu_sc as plsc

N_NODES = 10000
N_PAD = 10240
N_EDGES = 160000
E_PAD = 163840
D = 256
NUM_LAYERS = 4
NC, NS = 2, 16
DUMMY_COL = 10200

BLK = 512
GRID = N_PAD // BLK

CH = 1024
NCH = E_PAD // CH
FPP = 4


def _count_body(epk, out, cnt_a, cnt_b, pch, csp_a, csp_b, red_buf, res):
    cid = lax.axis_index("c")
    sid = lax.axis_index("s")
    wid = cid * NS + sid

    z16 = jnp.zeros((16,), jnp.float32)

    def zero_body(i, _):
        cnt_a[pl.ds(i * 16, 16)] = z16
        cnt_b[pl.ds(i * 16, 16)] = z16
        return 0
    lax.fori_loop(0, N_PAD // 16, zero_body, 0)

    ones = jnp.ones((16,), jnp.float32)
    e_per_tile = E_PAD // (NC * NS)

    def chunk_body(b, _):
        pltpu.sync_copy(epk.at[pl.ds(wid * e_per_tile + b * CH, CH)], pch)
        def vec_body(j, _):
            p = pch[pl.ds(j * 16, 16)]
            r = p & 0xFFFF
            c = p >> 16
            plsc.addupdate_scatter(cnt_a, [c], ones)
            plsc.addupdate_scatter(cnt_b, [c], ones, mask=r == c)
            return 0
        lax.fori_loop(0, CH // 16, vec_body, 0)
        return 0
    lax.fori_loop(0, e_per_tile // CH, chunk_body, 0)

    pltpu.sync_copy(cnt_a, csp_a.at[sid])
    pltpu.sync_copy(cnt_b, csp_b.at[sid])
    plsc.subcore_barrier()

    cols = N_PAD // NS
    cbase = sid * cols
    for which, csp in ((0, csp_a), (1, csp_b)):
        pltpu.sync_copy(csp.at[:, pl.ds(cbase, cols)], red_buf)
        def red_body(ch, _):
            acc = red_buf[0, pl.ds(ch * 16, 16)]
            for i in range(1, NS):
                acc = acc + red_buf[i, pl.ds(ch * 16, 16)]
            res[pl.ds(ch * 16, 16)] = acc
            return 0
        lax.fori_loop(0, cols // 16, red_body, 0)
        pltpu.sync_copy(res, out.at[pl.ds(which * 2 * N_PAD + cid * N_PAD
                                          + cbase, cols)])


def _count_call(epk):
    mesh = plsc.VectorSubcoreMesh(core_axis_name="c", subcore_axis_name="s")
    f = pl.kernel(
        _count_body,
        out_type=jax.ShapeDtypeStruct((4 * N_PAD,), jnp.float32),
        mesh=mesh,
        scratch_types=[
            pltpu.VMEM((N_PAD,), jnp.float32),
            pltpu.VMEM((N_PAD,), jnp.float32),
            pltpu.VMEM((CH,), jnp.int32),
            pltpu.VMEM_SHARED((NS, N_PAD), jnp.float32),
            pltpu.VMEM_SHARED((NS, N_PAD), jnp.float32),
            pltpu.VMEM((NS, N_PAD // NS), jnp.float32),
            pltpu.VMEM((N_PAD // NS,), jnp.float32),
        ],
        compiler_params=pltpu.CompilerParams(needs_layout_passes=False),
    )
    return f(epk)


def _fagg_body(gst, epk, st, g0, g1, g2, g3, a0, a1, a2, a3, pch, s0, s1):
    cid = lax.axis_index("c")
    sid = lax.axis_index("s")
    gr = (g0, g1, g2, g3)
    ar = (a0, a1, a2, a3)
    isems = (s0, s1)
    z16 = jnp.zeros((16,), jnp.float32)

    for ps in range(2):
        fb = cid * (D // 2) + sid * (2 * FPP) + ps * FPP
        for k in range(FPP):
            pltpu.sync_copy(gst.at[fb + k], gr[k])

        def zero_body(i, _):
            for k in range(FPP):
                ar[k][pl.ds(i * 16, 16)] = z16
            return 0
        lax.fori_loop(0, N_PAD // 16, zero_body, 0)

        pltpu.async_copy(epk.at[pl.ds(0, CH)], pch.at[0], isems[0])
        pltpu.async_copy(epk.at[pl.ds(CH, CH)], pch.at[1], isems[1])

        def outer(o, _):
            for s in range(2):
                ch = o * 2 + s
                pltpu.make_async_copy(epk.at[pl.ds(0, CH)], pch.at[s],
                                      isems[s]).wait()

                def vec_body(i, _):
                    for u in range(4):
                        p = pch[s, pl.ds(i * 64 + u * 16, 16)]
                        r = p & 0xFFFF
                        c = p >> 16
                        for k in range(FPP):
                            v = plsc.load_gather(gr[k], [r])
                            plsc.addupdate_scatter(ar[k], [c], v)
                    return 0
                lax.fori_loop(0, CH // 64, vec_body, 0)

                @pl.when(ch + 2 < NCH)
                def _():
                    pltpu.async_copy(epk.at[pl.ds((ch + 2) * CH, CH)],
                                     pch.at[s], isems[s])
            return 0
        lax.fori_loop(0, NCH // 2, outer, 0)

        for k in range(FPP):
            pltpu.sync_copy(ar[k], st.at[fb + k])


def _fagg_call(gst, epk):
    mesh = plsc.VectorSubcoreMesh(core_axis_name="c", subcore_axis_name="s")
    f = pl.kernel(
        _fagg_body,
        out_type=jax.ShapeDtypeStruct((D, N_PAD), jnp.float32),
        mesh=mesh,
        scratch_types=(
            [pltpu.VMEM((N_PAD,), jnp.float32)] * FPP
            + [pltpu.VMEM((N_PAD,), jnp.float32)] * FPP
            + [pltpu.VMEM((2, CH), jnp.int32)]
            + [pltpu.SemaphoreType.DMA] * 2
        ),
        compiler_params=pltpu.CompilerParams(needs_layout_passes=False),
    )
    return f(gst, epk)


def _gelu(v):
    return 0.5 * v * (1.0 + lax.erf(v * (2.0 ** -0.5)))


def _prep_body(pw_ref, aw_ref, wsym_ref, mt_ref):
    P = pw_ref[...]
    A = aw_ref[...]
    ri = lax.broadcasted_iota(jnp.int32, (D, D), 0)
    ci = lax.broadcasted_iota(jnp.int32, (D, D), 1)
    W0u = jnp.where(ci > ri, P[:, :D], 0.0)
    W0 = W0u + W0u.T
    rowsum = jnp.sum(jnp.abs(W0), axis=1, keepdims=True)
    q = P[:, D:D + 1]
    r = P[:, D + 1:D + 2]
    diag = q * rowsum + r
    wsym_ref[...] = W0 + jnp.where(ci == ri, diag, 0.0)
    mt_ref[...] = A - A.T


def _prep_call(P_W, A_W):
    return pl.pallas_call(
        _prep_body,
        out_shape=(jax.ShapeDtypeStruct((D, D), jnp.float32),
                   jax.ShapeDtypeStruct((D, D), jnp.float32)),
    )(P_W, A_W)


def _mm(a, b):
    return jnp.dot(a, b, preferred_element_type=jnp.float32)


def _head_body(xt_ref, wp_ref, bp_ref, wsym_ref, mt_ref, ab_ref,
               ca0_ref, ca1_ref, cb0_ref, cb1_ref,
               ht_ref, gt_ref, gst_ref, ut_ref, dis_ref, selfc_ref):
    indeg = ca0_ref[...] + ca1_ref[...]
    slc = cb0_ref[...] + cb1_ref[...]
    deg = indeg - slc
    dis = jnp.where(deg > 0, lax.rsqrt(deg), 0.0)
    dis_ref[...] = dis
    selfc_ref[...] = dis * dis * slc

    h = _gelu(_mm(wp_ref[...], xt_ref[...]) + bp_ref[...])
    ht_ref[...] = h
    g = _mm(wsym_ref[...], h)
    gt_ref[...] = g
    gst_ref[...] = dis * g
    ut_ref[...] = _mm(mt_ref[...], h) + ab_ref[...]


def _head_call(xt, Wp, bpc, Wsym, MT, Abc, ca0, ca1, cb0, cb1):
    full = pl.BlockSpec((D, D), lambda i: (0, 0))
    colv = pl.BlockSpec((D, 1), lambda i: (0, 0))
    nblk = pl.BlockSpec((D, BLK), lambda i: (0, i))
    cblk = pl.BlockSpec((1, BLK), lambda i: (0, i))
    return pl.pallas_call(
        _head_body,
        grid=(GRID,),
        in_specs=[nblk, full, colv, full, full, colv, cblk, cblk, cblk, cblk],
        out_specs=[nblk, nblk, nblk, nblk, cblk, cblk],
        out_shape=(jax.ShapeDtypeStruct((D, N_PAD), jnp.float32),
                   jax.ShapeDtypeStruct((D, N_PAD), jnp.float32),
                   jax.ShapeDtypeStruct((D, N_PAD), jnp.float32),
                   jax.ShapeDtypeStruct((D, N_PAD), jnp.float32),
                   jax.ShapeDtypeStruct((1, N_PAD), jnp.float32),
                   jax.ShapeDtypeStruct((1, N_PAD), jnp.float32)),
    )(xt, Wp, bpc, Wsym, MT, Abc, ca0, ca1, cb0, cb1)


def _layer_body(ht_ref, gt_ref, ut_ref, st_ref, dis_ref, selfc_ref,
                wsym_ref, mt_ref, ab_ref,
                hn_ref, gn_ref, gst_ref, un_ref):
    dis = dis_ref[...]
    agg = dis * st_ref[...] - selfc_ref[...] * gt_ref[...]
    hn = ht_ref[...] + _gelu(agg - ut_ref[...])
    hn_ref[...] = hn
    gn = _mm(wsym_ref[...], hn)
    gn_ref[...] = gn
    gst_ref[...] = dis * gn
    un_ref[...] = _mm(mt_ref[...], hn) + ab_ref[...]


def _layer_call(ht, gt, ut, st, dis, selfc, Wsym, MT, Abc):
    full = pl.BlockSpec((D, D), lambda i: (0, 0))
    colv = pl.BlockSpec((D, 1), lambda i: (0, 0))
    nblk = pl.BlockSpec((D, BLK), lambda i: (0, i))
    cblk = pl.BlockSpec((1, BLK), lambda i: (0, i))
    return pl.pallas_call(
        _layer_body,
        grid=(GRID,),
        in_specs=[nblk, nblk, nblk, nblk, cblk, cblk, full, full, colv],
        out_specs=[nblk, nblk, nblk, nblk],
        out_shape=(jax.ShapeDtypeStruct((D, N_PAD), jnp.float32),
                   jax.ShapeDtypeStruct((D, N_PAD), jnp.float32),
                   jax.ShapeDtypeStruct((D, N_PAD), jnp.float32),
                   jax.ShapeDtypeStruct((D, N_PAD), jnp.float32)),
    )(ht, gt, ut, st, dis, selfc, Wsym, MT, Abc)


def _tail_body(ht_ref, gt_ref, ut_ref, st_ref, dis_ref, selfc_ref, hn_ref):
    agg = dis_ref[...] * st_ref[...] - selfc_ref[...] * gt_ref[...]
    hn_ref[...] = ht_ref[...] + _gelu(agg - ut_ref[...])


def _tail_call(ht, gt, ut, st, dis, selfc):
    nblk = pl.BlockSpec((D, BLK), lambda i: (0, i))
    cblk = pl.BlockSpec((1, BLK), lambda i: (0, i))
    return pl.pallas_call(
        _tail_body,
        grid=(GRID,),
        in_specs=[nblk, nblk, nblk, nblk, cblk, cblk],
        out_specs=nblk,
        out_shape=jax.ShapeDtypeStruct((D, N_PAD), jnp.float32),
    )(ht, gt, ut, st, dis, selfc)


def kernel(x, edge_index, Wp, bp, A_W, A_b, P_W):
    row = edge_index[0].astype(jnp.int32)
    col = edge_index[1].astype(jnp.int32)
    npad = E_PAD - N_EDGES
    rowp = jnp.concatenate([row, jnp.zeros((npad,), jnp.int32)])
    colp = jnp.concatenate([col, jnp.full((npad,), DUMMY_COL, jnp.int32)])
    epk = rowp | (colp << 16)

    xt = jnp.pad(x, ((0, N_PAD - N_NODES), (0, 0))).T
    bpc = bp.reshape(D, 1)
    Abc = A_b.reshape(D, 1)

    Wsym, MT = _prep_call(P_W, A_W)
    cnt = _count_call(epk)
    ca0 = cnt[0 * N_PAD:1 * N_PAD].reshape(1, N_PAD)
    ca1 = cnt[1 * N_PAD:2 * N_PAD].reshape(1, N_PAD)
    cb0 = cnt[2 * N_PAD:3 * N_PAD].reshape(1, N_PAD)
    cb1 = cnt[3 * N_PAD:4 * N_PAD].reshape(1, N_PAD)

    ht, gt, gst, ut, dis, selfc = _head_call(
        xt, Wp, bpc, Wsym, MT, Abc, ca0, ca1, cb0, cb1)

    for _ in range(NUM_LAYERS - 1):
        st = _fagg_call(gst, epk)
        ht, gt, gst, ut = _layer_call(ht, gt, ut, st, dis, selfc,
                                      Wsym, MT, Abc)
    st = _fagg_call(gst, epk)
    ht = _tail_call(ht, gt, ut, st, dis, selfc)
    return ht.T[:N_NODES]


# speedup vs baseline: 1.8897x; 1.8897x over previous
import jax
import jax.numpy as jnp
from jax import lax
from jax.experimental import pallas as pl
from jax.experimental.pallas import tpu as pltpu
from jax.experimental.pallas import tpu_sc as plsc

N_NODES = 10000
N_PAD = 10240
N_EDGES = 160000
E_PAD = 163840
D = 256
H = 128
NUM_LAYERS = 4
NC, NS = 2, 16
DUMMY_COL = 10200

BLK = 512
GRID = N_PAD // BLK

CH = 1024


def _count_body(epk, out, cnt_a, cnt_b, pch, csp_a, csp_b, red_buf, res):
    cid = lax.axis_index("c")
    sid = lax.axis_index("s")
    wid = cid * NS + sid

    z16 = jnp.zeros((16,), jnp.float32)

    def zero_body(i, _):
        cnt_a[pl.ds(i * 16, 16)] = z16
        cnt_b[pl.ds(i * 16, 16)] = z16
        return 0
    lax.fori_loop(0, N_PAD // 16, zero_body, 0)

    ones = jnp.ones((16,), jnp.float32)
    e_per_tile = E_PAD // (NC * NS)

    def chunk_body(b, _):
        pltpu.sync_copy(epk.at[pl.ds(wid * e_per_tile + b * CH, CH)], pch)
        def vec_body(j, _):
            p = pch[pl.ds(j * 16, 16)]
            r = p & 0xFFFF
            c = p >> 16
            plsc.addupdate_scatter(cnt_a, [c], ones)
            plsc.addupdate_scatter(cnt_b, [c], ones, mask=r == c)
            return 0
        lax.fori_loop(0, CH // 16, vec_body, 0)
        return 0
    lax.fori_loop(0, e_per_tile // CH, chunk_body, 0)

    pltpu.sync_copy(cnt_a, csp_a.at[sid])
    pltpu.sync_copy(cnt_b, csp_b.at[sid])
    plsc.subcore_barrier()

    cols = N_PAD // NS
    cbase = sid * cols
    for which, csp in ((0, csp_a), (1, csp_b)):
        pltpu.sync_copy(csp.at[:, pl.ds(cbase, cols)], red_buf)
        def red_body(ch, _):
            acc = red_buf[0, pl.ds(ch * 16, 16)]
            for i in range(1, NS):
                acc = acc + red_buf[i, pl.ds(ch * 16, 16)]
            res[pl.ds(ch * 16, 16)] = acc
            return 0
        lax.fori_loop(0, cols // 16, red_body, 0)
        pltpu.sync_copy(res, out.at[pl.ds(which * 2 * N_PAD + cid * N_PAD
                                          + cbase, cols)])


def _count_call(epk):
    mesh = plsc.VectorSubcoreMesh(core_axis_name="c", subcore_axis_name="s")
    f = pl.kernel(
        _count_body,
        out_type=jax.ShapeDtypeStruct((4 * N_PAD,), jnp.float32),
        mesh=mesh,
        scratch_types=[
            pltpu.VMEM((N_PAD,), jnp.float32),
            pltpu.VMEM((N_PAD,), jnp.float32),
            pltpu.VMEM((CH,), jnp.int32),
            pltpu.VMEM_SHARED((NS, N_PAD), jnp.float32),
            pltpu.VMEM_SHARED((NS, N_PAD), jnp.float32),
            pltpu.VMEM((NS, N_PAD // NS), jnp.float32),
            pltpu.VMEM((N_PAD // NS,), jnp.float32),
        ],
        compiler_params=pltpu.CompilerParams(needs_layout_passes=False),
    )
    return f(epk)


NBUF = 4
EB = 64
NBATCH = E_PAD // NS // EB
NOUT = NBATCH // NBUF
PBW = NBATCH * EB // 128


def _agg_body(gs0, gs1, eidx, s0, s1, aggs, pbuf, idxu, rows, *sems):
    cid = lax.axis_index("c")
    sid = lax.axis_index("s")
    gsems = sems[:NBUF]
    ssems = sems[NBUF:]

    pidx = pltpu.async_copy(eidx.at[sid], pbuf, gsems[0])

    z16 = jnp.zeros((16,), jnp.float32)
    def zrow(i, _):
        for j in range(H // 16):
            rows[0, i, pl.ds(j * 16, 16)] = z16
        return 0
    lax.fori_loop(0, EB, zrow, 0)

    rows_per_tile = N_PAD // NS
    def zspmem(k, _):
        pltpu.sync_copy(rows.at[0],
                        aggs.at[pl.ds(sid * rows_per_tile + k * EB, EB)])
        return 0
    lax.fori_loop(0, rows_per_tile // EB, zspmem, 0)
    pidx.wait()
    plsc.subcore_barrier()

    def gather_start(b, s):
        for j in range(EB // 16):
            p = pbuf[b // 2, pl.ds((b % 2) * EB + j * 16, 16)]
            idxu[s, 0, pl.ds(j * 16, 16)] = p & 0xFFFF
            idxu[s, 1, pl.ds(j * 16, 16)] = p >> 16

        @pl.when(cid == 0)
        def _():
            pltpu.async_copy(gs0.at[idxu.at[s, 0]], rows.at[s], gsems[s])

        @pl.when(cid == 1)
        def _():
            pltpu.async_copy(gs1.at[idxu.at[s, 0]], rows.at[s], gsems[s])

    def gather_drain(s):
        pltpu.make_async_copy(gs0.at[pl.ds(0, EB)], rows.at[s],
                              gsems[s]).wait()

    def scatter_drain(s):
        pltpu.make_async_copy(gs0.at[pl.ds(0, EB)], rows.at[s],
                              ssems[s]).wait()

    for s in range(NBUF):
        gather_start(s, s)

    def outer(o, _):
        for s in range(NBUF):
            v = o * NBUF + s
            gather_drain(s)
            pltpu.async_copy(rows.at[s], aggs.at[idxu.at[s, 1]], ssems[s],
                             add=True)
            scatter_drain(s)

            @pl.when(o < NOUT - 1)
            def _():
                gather_start(v + NBUF, s)
        return 0
    lax.fori_loop(0, NOUT, outer, 0)
    plsc.subcore_barrier()

    obase = sid * rows_per_tile

    @pl.when(cid == 0)
    def _():
        pltpu.sync_copy(aggs.at[pl.ds(obase, rows_per_tile)],
                        s0.at[pl.ds(obase, rows_per_tile)])

    @pl.when(cid == 1)
    def _():
        pltpu.sync_copy(aggs.at[pl.ds(obase, rows_per_tile)],
                        s1.at[pl.ds(obase, rows_per_tile)])


def _agg_call(gs0, gs1, eidx):
    mesh = plsc.VectorSubcoreMesh(core_axis_name="c", subcore_axis_name="s")
    f = pl.kernel(
        _agg_body,
        out_type=(jax.ShapeDtypeStruct((N_PAD, H), jnp.float32),
                  jax.ShapeDtypeStruct((N_PAD, H), jnp.float32)),
        mesh=mesh,
        scratch_types=(
            [pltpu.VMEM_SHARED((N_PAD, H), jnp.float32),
             pltpu.VMEM((PBW, 128), jnp.int32),
             pltpu.VMEM((NBUF, 2, EB), jnp.int32),
             pltpu.VMEM((NBUF, EB, H), jnp.float32)]
            + [pltpu.SemaphoreType.DMA] * (2 * NBUF)
        ),
    )
    return f(gs0, gs1, eidx)


def _gelu(v):
    return 0.5 * v * (1.0 + lax.erf(v * (2.0 ** -0.5)))


def _prep_body(pw_ref, aw_ref, wsym_ref, m_ref):
    P = pw_ref[...]
    A = aw_ref[...]
    ri = lax.broadcasted_iota(jnp.int32, (D, D), 0)
    ci = lax.broadcasted_iota(jnp.int32, (D, D), 1)
    W0u = jnp.where(ci > ri, P[:, :D], 0.0)
    W0 = W0u + W0u.T
    rowsum = jnp.sum(jnp.abs(W0), axis=1, keepdims=True)
    q = P[:, D:D + 1]
    r = P[:, D + 1:D + 2]
    diag = q * rowsum + r
    wsym_ref[...] = W0 + jnp.where(ci == ri, diag, 0.0)
    m_ref[...] = A.T - A


def _prep_call(P_W, A_W):
    return pl.pallas_call(
        _prep_body,
        out_shape=(jax.ShapeDtypeStruct((D, D), jnp.float32),
                   jax.ShapeDtypeStruct((D, D), jnp.float32)),
    )(P_W, A_W)


def _head_body(x_ref, wp_ref, bp_ref, wsym_ref, m_ref, ab_ref,
               ca0_ref, ca1_ref, cb0_ref, cb1_ref,
               h_ref, g_ref, gs0_ref, gs1_ref, u_ref, dis_ref, selfc_ref):
    indeg = ca0_ref[...] + ca1_ref[...]
    slc = cb0_ref[...] + cb1_ref[...]
    deg = indeg - slc
    dis = jnp.where(deg > 0, lax.rsqrt(deg), 0.0)
    dis_ref[...] = dis
    selfc_ref[...] = dis * dis * slc

    x = x_ref[...]
    pre = lax.dot_general(x, wp_ref[...], (((1,), (1,)), ((), ())),
                          preferred_element_type=jnp.float32) + bp_ref[...]
    h = _gelu(pre)
    h_ref[...] = h
    g = jnp.dot(h, wsym_ref[...], preferred_element_type=jnp.float32)
    g_ref[...] = g
    gs = dis * g
    gs0_ref[...] = gs[:, :H]
    gs1_ref[...] = gs[:, H:]
    u_ref[...] = jnp.dot(h, m_ref[...],
                         preferred_element_type=jnp.float32) + ab_ref[...]


def _head_call(x, Wp, bp2, Wsym, M, Ab2, ca0, ca1, cb0, cb1):
    full = pl.BlockSpec((D, D), lambda i: (0, 0))
    vec = pl.BlockSpec((1, D), lambda i: (0, 0))
    nblk = pl.BlockSpec((BLK, D), lambda i: (i, 0))
    hblk = pl.BlockSpec((BLK, H), lambda i: (i, 0))
    cblk = pl.BlockSpec((BLK, 1), lambda i: (i, 0))
    return pl.pallas_call(
        _head_body,
        grid=(GRID,),
        in_specs=[nblk, full, vec, full, full, vec, cblk, cblk, cblk, cblk],
        out_specs=[nblk, nblk, hblk, hblk, nblk, cblk, cblk],
        out_shape=(jax.ShapeDtypeStruct((N_PAD, D), jnp.float32),
                   jax.ShapeDtypeStruct((N_PAD, D), jnp.float32),
                   jax.ShapeDtypeStruct((N_PAD, H), jnp.float32),
                   jax.ShapeDtypeStruct((N_PAD, H), jnp.float32),
                   jax.ShapeDtypeStruct((N_PAD, D), jnp.float32),
                   jax.ShapeDtypeStruct((N_PAD, 1), jnp.float32),
                   jax.ShapeDtypeStruct((N_PAD, 1), jnp.float32)),
    )(x, Wp, bp2, Wsym, M, Ab2, ca0, ca1, cb0, cb1)


def _layer_body(h_ref, g_ref, u_ref, s0_ref, s1_ref, dis_ref, selfc_ref,
                wsym_ref, m_ref, ab_ref,
                hn_ref, gn_ref, gs0_ref, gs1_ref, un_ref):
    dis = dis_ref[...]
    S = jnp.concatenate([s0_ref[...], s1_ref[...]], axis=1)
    agg = dis * S - selfc_ref[...] * g_ref[...]
    hn = h_ref[...] + _gelu(agg - u_ref[...])
    hn_ref[...] = hn
    gn = jnp.dot(hn, wsym_ref[...], preferred_element_type=jnp.float32)
    gn_ref[...] = gn
    gs = dis * gn
    gs0_ref[...] = gs[:, :H]
    gs1_ref[...] = gs[:, H:]
    un_ref[...] = jnp.dot(hn, m_ref[...],
                          preferred_element_type=jnp.float32) + ab_ref[...]


def _layer_call(h, g, u, s0, s1, dis, selfc, Wsym, M, Ab2):
    full = pl.BlockSpec((D, D), lambda i: (0, 0))
    vec = pl.BlockSpec((1, D), lambda i: (0, 0))
    nblk = pl.BlockSpec((BLK, D), lambda i: (i, 0))
    hblk = pl.BlockSpec((BLK, H), lambda i: (i, 0))
    cblk = pl.BlockSpec((BLK, 1), lambda i: (i, 0))
    return pl.pallas_call(
        _layer_body,
        grid=(GRID,),
        in_specs=[nblk, nblk, nblk, hblk, hblk, cblk, cblk, full, full, vec],
        out_specs=[nblk, nblk, hblk, hblk, nblk],
        out_shape=(jax.ShapeDtypeStruct((N_PAD, D), jnp.float32),
                   jax.ShapeDtypeStruct((N_PAD, D), jnp.float32),
                   jax.ShapeDtypeStruct((N_PAD, H), jnp.float32),
                   jax.ShapeDtypeStruct((N_PAD, H), jnp.float32),
                   jax.ShapeDtypeStruct((N_PAD, D), jnp.float32)),
    )(h, g, u, s0, s1, dis, selfc, Wsym, M, Ab2)


def _tail_body(h_ref, g_ref, u_ref, s0_ref, s1_ref, dis_ref, selfc_ref,
               hn_ref):
    S = jnp.concatenate([s0_ref[...], s1_ref[...]], axis=1)
    agg = dis_ref[...] * S - selfc_ref[...] * g_ref[...]
    hn_ref[...] = h_ref[...] + _gelu(agg - u_ref[...])


def _tail_call(h, g, u, s0, s1, dis, selfc):
    nblk = pl.BlockSpec((BLK, D), lambda i: (i, 0))
    hblk = pl.BlockSpec((BLK, H), lambda i: (i, 0))
    cblk = pl.BlockSpec((BLK, 1), lambda i: (i, 0))
    return pl.pallas_call(
        _tail_body,
        grid=(GRID,),
        in_specs=[nblk, nblk, nblk, hblk, hblk, cblk, cblk],
        out_specs=nblk,
        out_shape=jax.ShapeDtypeStruct((N_PAD, D), jnp.float32),
    )(h, g, u, s0, s1, dis, selfc)


def kernel(x, edge_index, Wp, bp, A_W, A_b, P_W):
    row = edge_index[0].astype(jnp.int32)
    col = edge_index[1].astype(jnp.int32)
    npad = E_PAD - N_EDGES
    rowp = jnp.concatenate([row, jnp.zeros((npad,), jnp.int32)])
    colp = jnp.concatenate([col, jnp.full((npad,), DUMMY_COL, jnp.int32)])
    epk = rowp | (colp << 16)
    eidx = epk.reshape(NS, PBW, 128)

    xp = jnp.pad(x, ((0, N_PAD - N_NODES), (0, 0)))
    bp2 = bp.reshape(1, D)
    Ab2 = A_b.reshape(1, D)

    Wsym, M = _prep_call(P_W, A_W)
    cnt = _count_call(epk)
    ca0 = cnt[0 * N_PAD:1 * N_PAD].reshape(N_PAD, 1)
    ca1 = cnt[1 * N_PAD:2 * N_PAD].reshape(N_PAD, 1)
    cb0 = cnt[2 * N_PAD:3 * N_PAD].reshape(N_PAD, 1)
    cb1 = cnt[3 * N_PAD:4 * N_PAD].reshape(N_PAD, 1)

    h, g, gs0, gs1, u, dis, selfc = _head_call(
        xp, Wp, bp2, Wsym, M, Ab2, ca0, ca1, cb0, cb1)

    for _ in range(NUM_LAYERS - 1):
        s0, s1 = _agg_call(gs0, gs1, eidx)
        h, g, gs0, gs1, u = _layer_call(h, g, u, s0, s1, dis, selfc,
                                        Wsym, M, Ab2)
    s0, s1 = _agg_call(gs0, gs1, eidx)
    h = _tail_call(h, g, u, s0, s1, dis, selfc)
    return h[:N_NODES]
